# baseline (speedup 1.0000x reference)
import functools

import jax
import jax.numpy as jnp
from jax import lax
from jax.experimental import pallas as pl
from jax.experimental.pallas import tpu as pltpu
from jax.experimental.pallas import tpu_sc as plsc

_MAXLEN = 2048
_EMBED = 64
_BATCH = 64
_NW = 32
_ROWS = _BATCH * _MAXLEN
_RPW = _ROWS // _NW
_CHUNK = 512
_NCHUNK = _RPW // _CHUNK
_SUB = 128
_NSUB = _CHUNK // _SUB
_LANES = 16


@functools.partial(
    pl.kernel,
    mesh=plsc.VectorSubcoreMesh(core_axis_name="c", subcore_axis_name="s"),
    out_type=jax.ShapeDtypeStruct((_ROWS, _EMBED), jnp.float32),
    scratch_types=[
        pltpu.VMEM((_CHUNK,), jnp.int32),
        pltpu.VMEM((_CHUNK, _EMBED), jnp.float32),
        pltpu.VMEM((_CHUNK, _EMBED), jnp.float32),
        pltpu.SemaphoreType.DMA,
    ],
    compiler_params=pltpu.CompilerParams(use_tc_tiling_on_sc=False),
)
def _tokpos(x_hbm, tok_hbm, pos_hbm, out_hbm, idx_v, rows_v, pos_v, sem):
    wid = lax.axis_index("s") * 2 + lax.axis_index("c")
    base = wid * _RPW
    for c in range(_NCHUNK):
        gbase = base + c * _CHUNK
        pbase = (c * _CHUNK) % _MAXLEN
        pltpu.sync_copy(x_hbm.at[pl.ds(gbase, _CHUNK)], idx_v)
        pltpu.sync_copy(pos_hbm.at[pl.ds(pbase, _CHUNK)], pos_v)
        copies = [
            pltpu.async_copy(
                tok_hbm.at[idx_v.at[pl.ds(k * _SUB, _SUB)]],
                rows_v.at[pl.ds(k * _SUB, _SUB)],
                sem,
            )
            for k in range(_NSUB)
        ]
        for cp in copies:
            cp.wait()

        def body(r, carry):
            for e in range(_EMBED // _LANES):
                sl = pl.ds(e * _LANES, _LANES)
                rows_v[r, sl] = rows_v[r, sl] + pos_v[r, sl]
            return carry

        lax.fori_loop(0, _CHUNK, body, 0)
        pltpu.sync_copy(rows_v, out_hbm.at[pl.ds(gbase, _CHUNK)])


def kernel(x, token_table, pos_table):
    xf = x.reshape(-1).astype(jnp.int32)
    out = _tokpos(xf, token_table, pos_table)
    return out.reshape(x.shape[0], x.shape[1], _EMBED)
